# Initial kernel scaffold; baseline (speedup 1.0000x reference)
import jax
import jax.numpy as jnp
from jax.experimental import pallas as pl


def kernel(hidden_states, gate_weight, w1s, w2s, w3s):
    raise NotImplementedError("write your pallas kernel here")


# speedup vs baseline: 1.1855x; 1.1855x over previous
import functools

import jax
import jax.numpy as jnp
from jax import lax
from jax.experimental import pallas as pl
from jax.experimental.pallas import tpu as pltpu

E = 8
TOPK = 2
H = 1024
I = 2048
S = 2048
P = S * TOPK

BM = 512
BN = 512
NB = P // BM
NN = I // BN
MAX_STEPS = NB + E - 1


def _router_body(x_ref, g_ref, ids_ref, rw_ref):
    x = x_ref[...]
    g = g_ref[...]
    logits = lax.dot_general(x, g, (((1,), (1,)), ((), ())),
                             preferred_element_type=jnp.float32)
    m = jnp.max(logits, axis=1, keepdims=True)
    p = jnp.exp(logits - m)
    probs = p / jnp.sum(p, axis=1, keepdims=True)

    cols = lax.broadcasted_iota(jnp.int32, (S, E), 1)
    v1 = jnp.max(probs, axis=1, keepdims=True)
    i1 = jnp.min(jnp.where(probs == v1, cols, E), axis=1, keepdims=True)
    probs2 = jnp.where(cols == i1, -jnp.inf, probs)
    v2 = jnp.max(probs2, axis=1, keepdims=True)
    i2 = jnp.min(jnp.where(probs2 == v2, cols, E), axis=1, keepdims=True)

    denom = v1 + v2
    ids_ref[:, 0:1] = i1
    ids_ref[:, 1:2] = i2
    rw_ref[:, 0:1] = v1 / denom
    rw_ref[:, 1:2] = v2 / denom


def _router(x, gate_weight):
    return pl.pallas_call(
        _router_body,
        out_shape=(
            jax.ShapeDtypeStruct((S, TOPK), jnp.int32),
            jax.ShapeDtypeStruct((S, TOPK), jnp.float32),
        ),
    )(x, gate_weight)


def _gemm_body(se_ref, sm_ref, sf_ref, slo_ref, shi_ref,
               x_ref, w1_ref, w3_ref, w2_ref, rw_ref, out_ref):
    i = pl.program_id(0)
    n = pl.program_id(1)
    first = sf_ref[i]
    lo = slo_ref[i]
    hi = shi_ref[i]

    x = x_ref[...]
    h1 = jnp.dot(x, w1_ref[0], preferred_element_type=jnp.float32)
    h3 = jnp.dot(x, w3_ref[0], preferred_element_type=jnp.float32)
    prod = (h1 * jax.nn.sigmoid(h1)) * h3
    acc = jnp.dot(prod, w2_ref[0], preferred_element_type=jnp.float32)

    rows = sm_ref[i] * BM + lax.broadcasted_iota(jnp.int32, (BM, 1), 0)
    mask = (rows >= lo) & (rows < hi)
    acc = jnp.where(mask, acc * rw_ref[...], 0.0)

    @pl.when((n == 0) & (first == 1))
    def _():
        out_ref[...] = jnp.zeros_like(out_ref)

    out_ref[...] += acc


def _grouped_gemm(x_sorted, rw_sorted, w1s, w3s, w2s, se, sm, sf, slo, shi):
    grid_spec = pltpu.PrefetchScalarGridSpec(
        num_scalar_prefetch=5,
        grid=(MAX_STEPS, NN),
        in_specs=[
            pl.BlockSpec((BM, H), lambda i, n, se, sm, sf, lo, hi: (sm[i], 0)),
            pl.BlockSpec((1, H, BN), lambda i, n, se, sm, sf, lo, hi: (se[i], 0, n)),
            pl.BlockSpec((1, H, BN), lambda i, n, se, sm, sf, lo, hi: (se[i], 0, n)),
            pl.BlockSpec((1, BN, H), lambda i, n, se, sm, sf, lo, hi: (se[i], n, 0)),
            pl.BlockSpec((BM, 1), lambda i, n, se, sm, sf, lo, hi: (sm[i], 0)),
        ],
        out_specs=pl.BlockSpec((BM, H), lambda i, n, se, sm, sf, lo, hi: (sm[i], 0)),
    )
    return pl.pallas_call(
        _gemm_body,
        grid_spec=grid_spec,
        out_shape=jax.ShapeDtypeStruct((P, H), jnp.float32),
    )(se, sm, sf, slo, shi, x_sorted, w1s, w3s, w2s, rw_sorted)


def kernel(hidden_states, gate_weight, w1s, w2s, w3s):
    b, s, h = hidden_states.shape
    x = hidden_states.reshape(S, H)

    ids, rw = _router(x, gate_weight)

    flat_e = ids.reshape(P)
    onehot = (flat_e[:, None] == jnp.arange(E, dtype=jnp.int32)[None, :])
    onehot = onehot.astype(jnp.int32)
    counts = jnp.sum(onehot, axis=0)
    offs = jnp.concatenate([jnp.zeros((1,), jnp.int32),
                            jnp.cumsum(counts, dtype=jnp.int32)])
    within = jnp.cumsum(onehot, axis=0, dtype=jnp.int32) - onehot
    rank = offs[flat_e] + jnp.take_along_axis(within, flat_e[:, None], 1)[:, 0]

    token_of_pair = jnp.arange(P, dtype=jnp.int32) // TOPK
    row_src = jnp.zeros((P,), jnp.int32).at[rank].set(token_of_pair)
    rw_sorted = jnp.zeros((P,), jnp.float32).at[rank].set(rw.reshape(P))

    m_idx = (jnp.arange(NB * E, dtype=jnp.int32) // E)
    e_idx = (jnp.arange(NB * E, dtype=jnp.int32) % E)
    blk_lo = m_idx * BM
    valid = (offs[e_idx] < blk_lo + BM) & (offs[e_idx + 1] > blk_lo)
    key = jnp.where(valid, 0, NB * E) + jnp.arange(NB * E, dtype=jnp.int32)
    perm = jnp.argsort(key)
    nvalid = jnp.sum(valid.astype(jnp.int32))
    sel = perm[:MAX_STEPS]
    last = perm[jnp.maximum(nvalid - 1, 0)]
    step_live = jnp.arange(MAX_STEPS, dtype=jnp.int32) < nvalid
    sel = jnp.where(step_live, sel, last)
    se = e_idx[sel]
    sm = m_idx[sel]
    sf = (step_live & ((jnp.arange(MAX_STEPS) == 0) | (jnp.roll(sm, 1) != sm)))
    sf = sf.astype(jnp.int32)
    slo = jnp.where(step_live, offs[se], 0)
    shi = jnp.where(step_live, offs[se + 1], 0)

    x_sorted = jnp.take(x, row_src, axis=0)

    out_sorted = _grouped_gemm(x_sorted, rw_sorted[:, None], w1s, w3s, w2s,
                               se, sm, sf, slo, shi)

    pos = rank.reshape(S, TOPK)
    final = (jnp.take(out_sorted, pos[:, 0], axis=0)
             + jnp.take(out_sorted, pos[:, 1], axis=0))
    return final.reshape(b, s, h)


# speedup vs baseline: 1.2965x; 1.0937x over previous
import functools

import jax
import jax.numpy as jnp
from jax import lax
from jax.experimental import pallas as pl
from jax.experimental.pallas import tpu as pltpu

E = 8
TOPK = 2
H = 1024
I = 2048
S = 2048
P = S * TOPK

BM = 512
BN = 512
NB = P // BM
NN = I // BN
MAX_STEPS = NB + E - 1

PA = 128
PB = P // PA


def _tri(n, dtype=jnp.float32):
    r = lax.broadcasted_iota(jnp.int32, (n, n), 0)
    c = lax.broadcasted_iota(jnp.int32, (n, n), 1)
    return (r < c).astype(dtype)


def _router_body(x_ref, g_ref, rw_ref, rank_ref, st_ref):
    x = x_ref[...]
    g = g_ref[...]
    logits = lax.dot_general(x, g, (((1,), (1,)), ((), ())),
                             preferred_element_type=jnp.float32)
    m = jnp.max(logits, axis=1, keepdims=True)
    p = jnp.exp(logits - m)
    probs = p / jnp.sum(p, axis=1, keepdims=True)

    cols = lax.broadcasted_iota(jnp.int32, (S, E), 1)
    v1 = jnp.max(probs, axis=1, keepdims=True)
    i1 = jnp.min(jnp.where(probs == v1, cols, E), axis=1, keepdims=True)
    probs2 = jnp.where(cols == i1, -jnp.inf, probs)
    v2 = jnp.max(probs2, axis=1, keepdims=True)
    i2 = jnp.min(jnp.where(probs2 == v2, cols, E), axis=1, keepdims=True)

    denom = v1 + v2
    rw_ref[:, 0:1] = v1 / denom
    rw_ref[:, 1:2] = v2 / denom

    eidx = lax.broadcasted_iota(jnp.int32, (S, E), 1)
    oh1 = (eidx == i1).astype(jnp.float32)
    oh2 = (eidx == i2).astype(jnp.float32)
    ohsum = oh1 + oh2
    bpre = lax.dot_general(_tri(S), ohsum, (((0,), (0,)), ((), ())),
                           preferred_element_type=jnp.float32)
    counts = jnp.sum(ohsum, axis=0, keepdims=True)
    offs = lax.dot_general(counts, _tri(E), (((1,), (0,)), ((), ())),
                           preferred_element_type=jnp.float32)

    basef = offs + bpre
    rank_ref[:, 0:1] = jnp.sum(oh1 * basef, axis=1,
                               keepdims=True).astype(jnp.int32)
    rank_ref[:, 1:2] = jnp.sum(oh2 * basef, axis=1,
                               keepdims=True).astype(jnp.int32)

    offs_v = offs[0]
    ends_v = offs_v + counts[0]
    mlo = lax.broadcasted_iota(jnp.int32, (NB, E), 0).astype(jnp.float32) * BM
    valid = (offs_v[None, :] < mlo + BM) & (ends_v[None, :] > mlo)
    validf = valid.astype(jnp.float32)
    rowcnt = jnp.sum(validf, axis=1, keepdims=True)
    rowpre = lax.dot_general(_tri(NB), rowcnt, (((0,), (0,)), ((), ())),
                             preferred_element_type=jnp.float32)
    colpre = lax.dot_general(validf, _tri(E), (((1,), (0,)), ((), ())),
                             preferred_element_type=jnp.float32)
    pos = rowpre + colpre
    eidx2 = lax.broadcasted_iota(jnp.int32, (NB, E), 1).astype(jnp.float32)
    midx2 = lax.broadcasted_iota(jnp.int32, (NB, E), 0).astype(jnp.float32)
    e_last = jnp.max(jnp.where(valid, eidx2, -1.0))

    prev_sm = jnp.float32(-1.0)
    for s in range(MAX_STEPS):
        sel = validf * (pos == s).astype(jnp.float32)
        live = jnp.sum(sel) > 0.5
        se_s = jnp.where(live, jnp.sum(eidx2 * sel), e_last)
        sm_s = jnp.where(live, jnp.sum(midx2 * sel), float(NB - 1))
        lo_s = jnp.where(live, jnp.sum(offs_v[None, :] * sel), 0.0)
        hi_s = jnp.where(live, jnp.sum(ends_v[None, :] * sel), 0.0)
        sf_s = jnp.where(live & (sm_s != prev_sm), 1.0, 0.0)
        prev_sm = sm_s
        st_ref[0, s] = se_s.astype(jnp.int32)
        st_ref[1, s] = sm_s.astype(jnp.int32)
        st_ref[2, s] = sf_s.astype(jnp.int32)
        st_ref[3, s] = lo_s.astype(jnp.int32)
        st_ref[4, s] = hi_s.astype(jnp.int32)


def _router(x, gate_weight):
    return pl.pallas_call(
        _router_body,
        out_shape=(
            jax.ShapeDtypeStruct((S, TOPK), jnp.float32),
            jax.ShapeDtypeStruct((S, TOPK), jnp.int32),
            jax.ShapeDtypeStruct((5, MAX_STEPS), jnp.int32),
        ),
        out_specs=(
            pl.BlockSpec((S, TOPK), lambda: (0, 0)),
            pl.BlockSpec((S, TOPK), lambda: (0, 0)),
            pl.BlockSpec(memory_space=pltpu.SMEM),
        ),
    )(x, gate_weight)


def _gemm_body(se_ref, sm_ref, sf_ref, slo_ref, shi_ref,
               x_ref, w1_ref, w3_ref, w2_ref, rw_ref, out_ref):
    i = pl.program_id(0)
    n = pl.program_id(1)
    first = sf_ref[i]
    lo = slo_ref[i]
    hi = shi_ref[i]

    x = x_ref[...]
    h1 = jnp.dot(x, w1_ref[0], preferred_element_type=jnp.float32)
    h3 = jnp.dot(x, w3_ref[0], preferred_element_type=jnp.float32)
    prod = (h1 * jax.nn.sigmoid(h1)) * h3
    acc = jnp.dot(prod, w2_ref[0], preferred_element_type=jnp.float32)

    rows = sm_ref[i] * BM + lax.broadcasted_iota(jnp.int32, (BM, 1), 0)
    mask = (rows >= lo) & (rows < hi)
    acc = jnp.where(mask, acc * rw_ref[...], 0.0)

    @pl.when((n == 0) & (first == 1))
    def _():
        out_ref[...] = jnp.zeros_like(out_ref)

    out_ref[...] += acc


def _grouped_gemm(x_sorted, rw_sorted, w1s, w3s, w2s, se, sm, sf, slo, shi):
    grid_spec = pltpu.PrefetchScalarGridSpec(
        num_scalar_prefetch=5,
        grid=(MAX_STEPS, NN),
        in_specs=[
            pl.BlockSpec((BM, H), lambda i, n, se, sm, sf, lo, hi: (sm[i], 0)),
            pl.BlockSpec((1, H, BN), lambda i, n, se, sm, sf, lo, hi: (se[i], 0, n)),
            pl.BlockSpec((1, H, BN), lambda i, n, se, sm, sf, lo, hi: (se[i], 0, n)),
            pl.BlockSpec((1, BN, H), lambda i, n, se, sm, sf, lo, hi: (se[i], n, 0)),
            pl.BlockSpec((BM, 1), lambda i, n, se, sm, sf, lo, hi: (sm[i], 0)),
        ],
        out_specs=pl.BlockSpec((BM, H), lambda i, n, se, sm, sf, lo, hi: (sm[i], 0)),
    )
    return pl.pallas_call(
        _gemm_body,
        grid_spec=grid_spec,
        out_shape=jax.ShapeDtypeStruct((P, H), jnp.float32),
    )(se, sm, sf, slo, shi, x_sorted, w1s, w3s, w2s, rw_sorted)


def kernel(hidden_states, gate_weight, w1s, w2s, w3s):
    b, s, h = hidden_states.shape
    x = hidden_states.reshape(S, H)

    rw, rank2d, st = _router(x, gate_weight)
    rank = rank2d.reshape(P)
    se, sm, sf, slo, shi = st[0], st[1], st[2], st[3], st[4]

    token_of_pair = jnp.arange(P, dtype=jnp.int32) // TOPK
    row_src = jnp.zeros((P,), jnp.int32).at[rank].set(token_of_pair)
    rw_sorted = jnp.zeros((P,), jnp.float32).at[rank].set(rw.reshape(P))

    x_sorted = jnp.take(x, row_src, axis=0)

    out_sorted = _grouped_gemm(x_sorted, rw_sorted[:, None], w1s, w3s, w2s,
                               se, sm, sf, slo, shi)

    pos = rank.reshape(S, TOPK)
    final = (jnp.take(out_sorted, pos[:, 0], axis=0)
             + jnp.take(out_sorted, pos[:, 1], axis=0))
    return final.reshape(b, s, h)
